# baseline (speedup 1.0000x reference)
import functools

import jax
import jax.numpy as jnp
from jax import lax
from jax.experimental import pallas as pl
from jax.experimental.pallas import tpu as pltpu
from jax.experimental.pallas import tpu_sc as plsc

O = 2048
IL = 4096
K = 32
B = 1024

NC = 2
NS = 16
NW = NC * NS
NSTAGE = 1
OS = O // NSTAGE
RPT = OS // NW
C = 8
NCH = RPT // C


def _sc_body(w_hbm, idx_hbm, z_hbm, mt_hbm, idx_v, w_v, buf0, buf1, sem0, sem1):
    wid = lax.axis_index("c") * NS + lax.axis_index("s")
    rbase = wid * RPT

    h0 = pltpu.async_copy(z_hbm, buf0, sem0)
    h1 = pltpu.async_copy(z_hbm, buf1, sem1)
    pltpu.sync_copy(idx_hbm.at[pl.ds(rbase, RPT)], idx_v)
    pltpu.sync_copy(w_hbm.at[pl.ds(rbase, RPT)], w_v)
    h0.wait()
    h1.wait()

    z16 = jnp.zeros((16,), jnp.float32)
    bufs = (buf0, buf1)
    sems = (sem0, sem1)

    def _for_chunk(c, fn):
        b = bufs[c % 2]
        for j in range(C):
            r = c * C + j
            jv = jnp.full((16,), j, jnp.int32)
            for h in range(2):
                fn(b, [jv, idx_v[r, pl.ds(h * 16, 16)]], w_v[r, pl.ds(h * 16, 16)])

    handles = [None, None]
    for c in range(NCH):
        s = c % 2
        if handles[s] is not None:
            handles[s].wait()
            _for_chunk(c - 2, lambda b, ix, wv: plsc.store_scatter(b, ix, z16))
        _for_chunk(c, lambda b, ix, wv: plsc.addupdate_scatter(b, ix, wv))
        dst = mt_hbm.at[pl.ds(rbase + c * C, C)]
        handles[s] = pltpu.async_copy(bufs[s], dst, sems[s])
    handles[0].wait()
    handles[1].wait()


_build_mt = pl.kernel(
    _sc_body,
    out_type=jax.ShapeDtypeStruct((OS, IL), jnp.float32),
    mesh=plsc.VectorSubcoreMesh(
        core_axis_name="c", subcore_axis_name="s", num_cores=NC, num_subcores=NS
    ),
    compiler_params=pltpu.CompilerParams(needs_layout_passes=False),
    scratch_types=[
        pltpu.VMEM((RPT, K), jnp.int32),
        pltpu.VMEM((RPT, K), jnp.float32),
        pltpu.VMEM((C, IL), jnp.float32),
        pltpu.VMEM((C, IL), jnp.float32),
        pltpu.SemaphoreType.DMA,
        pltpu.SemaphoreType.DMA,
    ],
)


BO = 512


def _mm_body(x_ref, mt_ref, b_ref, o_ref):
    acc = lax.dot_general(
        x_ref[...].astype(jnp.bfloat16), mt_ref[...].astype(jnp.bfloat16),
        (((1,), (1,)), ((), ())),
        preferred_element_type=jnp.float32,
    )
    o_ref[...] = acc + b_ref[...]


def _matmul(x, mt, bias2d):
    return pl.pallas_call(
        _mm_body,
        grid=(OS // BO,),
        in_specs=[
            pl.BlockSpec((B, IL), lambda i: (0, 0)),
            pl.BlockSpec((BO, IL), lambda i: (i, 0)),
            pl.BlockSpec((1, BO), lambda i: (0, i)),
        ],
        out_specs=pl.BlockSpec((B, BO), lambda i: (0, i)),
        out_shape=jax.ShapeDtypeStruct((B, OS), jnp.float32),
    )(x, mt, bias2d)


def kernel(input, weight, bias, indx_seqs):
    zeros = jnp.zeros((C, IL), jnp.float32)
    mt = _build_mt(weight, indx_seqs, zeros)
    return _matmul(input, mt, bias.reshape(1, O))


# speedup vs baseline: 1.1433x; 1.1433x over previous
import functools

import jax
import jax.numpy as jnp
from jax import lax
from jax.experimental import pallas as pl
from jax.experimental.pallas import tpu as pltpu
from jax.experimental.pallas import tpu_sc as plsc

O = 2048
IL = 4096
K = 32
B = 1024

NC = 2
NS = 16
NW = NC * NS
NSTAGE = 1
OS = O // NSTAGE
RPT = OS // NW
C = 8
NCH = RPT // C


def _sc_body(w_hbm, idx_hbm, mt_hbm, idx_v, w_v, buf0, buf1, sem0, sem1):
    wid = lax.axis_index("c") * NS + lax.axis_index("s")
    rbase = wid * RPT
    pltpu.sync_copy(idx_hbm.at[pl.ds(rbase, RPT)], idx_v)
    pltpu.sync_copy(w_hbm.at[pl.ds(rbase, RPT)], w_v)

    z16 = jnp.zeros((16,), jnp.float32)
    bufs = (buf0, buf1)
    sems = (sem0, sem1)

    def _memset(t, carry):
        for j in range(C):
            buf0[j, pl.ds(t * 16, 16)] = z16
            buf1[j, pl.ds(t * 16, 16)] = z16
        return carry
    lax.fori_loop(0, IL // 16, _memset, 0)

    def _for_chunk(c, fn):
        b = bufs[c % 2]
        for j in range(C):
            r = c * C + j
            jv = jnp.full((16,), j, jnp.int32)
            for h in range(2):
                fn(b, [jv, idx_v[r, pl.ds(h * 16, 16)]], w_v[r, pl.ds(h * 16, 16)])

    handles = [None, None]
    for c in range(NCH):
        s = c % 2
        if handles[s] is not None:
            handles[s].wait()
            _for_chunk(c - 2, lambda b, ix, wv: plsc.store_scatter(b, ix, z16))
        _for_chunk(c, lambda b, ix, wv: plsc.addupdate_scatter(b, ix, wv))
        dst = mt_hbm.at[pl.ds(rbase + c * C, C)]
        handles[s] = pltpu.async_copy(bufs[s], dst, sems[s])
    handles[0].wait()
    handles[1].wait()


_build_mt = pl.kernel(
    _sc_body,
    out_type=jax.ShapeDtypeStruct((OS, IL), jnp.float32),
    mesh=plsc.VectorSubcoreMesh(
        core_axis_name="c", subcore_axis_name="s", num_cores=NC, num_subcores=NS
    ),
    compiler_params=pltpu.CompilerParams(needs_layout_passes=False),
    scratch_types=[
        pltpu.VMEM((RPT, K), jnp.int32),
        pltpu.VMEM((RPT, K), jnp.float32),
        pltpu.VMEM((C, IL), jnp.float32),
        pltpu.VMEM((C, IL), jnp.float32),
        pltpu.SemaphoreType.DMA,
        pltpu.SemaphoreType.DMA,
    ],
)


BO = 512


def _mm_body(x_ref, mt_ref, b_ref, o_ref):
    acc = lax.dot_general(
        x_ref[...].astype(jnp.bfloat16), mt_ref[...].astype(jnp.bfloat16),
        (((1,), (1,)), ((), ())),
        preferred_element_type=jnp.float32,
    )
    o_ref[...] = acc + b_ref[...]


def _matmul(x, mt, bias2d):
    return pl.pallas_call(
        _mm_body,
        grid=(OS // BO,),
        in_specs=[
            pl.BlockSpec((B, IL), lambda i: (0, 0)),
            pl.BlockSpec((BO, IL), lambda i: (i, 0)),
            pl.BlockSpec((1, BO), lambda i: (0, i)),
        ],
        out_specs=pl.BlockSpec((B, BO), lambda i: (0, i)),
        out_shape=jax.ShapeDtypeStruct((B, OS), jnp.float32),
    )(x, mt, bias2d)


def kernel(input, weight, bias, indx_seqs):
    mt = _build_mt(weight, indx_seqs)
    return _matmul(input, mt, bias.reshape(1, O))


# speedup vs baseline: 1.1919x; 1.0425x over previous
import functools

import jax
import jax.numpy as jnp
from jax import lax
from jax.experimental import pallas as pl
from jax.experimental.pallas import tpu as pltpu
from jax.experimental.pallas import tpu_sc as plsc

O = 2048
IL = 4096
K = 32
B = 1024

NC = 2
NS = 16
NW = NC * NS
NSTAGE = 1
OS = O // NSTAGE
RPT = OS // NW
C = 4
NCH = RPT // C


def _sc_body(w_hbm, idx_hbm, mt_hbm, idx_v, w_v, buf0, buf1, sem0, sem1):
    wid = lax.axis_index("c") * NS + lax.axis_index("s")
    rbase = wid * RPT
    hi = pltpu.async_copy(idx_hbm.at[pl.ds(rbase, RPT)], idx_v, sem0)
    hw = pltpu.async_copy(w_hbm.at[pl.ds(rbase, RPT)], w_v, sem1)

    z16 = jnp.zeros((16,), jnp.float32)
    bufs = (buf0, buf1)
    sems = (sem0, sem1)

    def _memset(t, carry):
        for j in range(C):
            buf0[j, pl.ds(t * 16, 16)] = z16
            buf1[j, pl.ds(t * 16, 16)] = z16
        return carry
    lax.fori_loop(0, IL // 16, _memset, 0)
    hi.wait()
    hw.wait()

    def _for_chunk(c, fn):
        b = bufs[c % 2]
        for j in range(C):
            r = c * C + j
            jv = jnp.full((16,), j, jnp.int32)
            for h in range(2):
                fn(b, [jv, idx_v[r, pl.ds(h * 16, 16)]], w_v[r, pl.ds(h * 16, 16)])

    handles = [None, None]
    for c in range(NCH):
        s = c % 2
        if handles[s] is not None:
            handles[s].wait()
            _for_chunk(c - 2, lambda b, ix, wv: plsc.store_scatter(b, ix, z16))
        _for_chunk(c, lambda b, ix, wv: plsc.addupdate_scatter(b, ix, wv))
        dst = mt_hbm.at[pl.ds(rbase + c * C, C)]
        handles[s] = pltpu.async_copy(bufs[s], dst, sems[s])
    handles[0].wait()
    handles[1].wait()


_build_mt = pl.kernel(
    _sc_body,
    out_type=jax.ShapeDtypeStruct((OS, IL), jnp.float32),
    mesh=plsc.VectorSubcoreMesh(
        core_axis_name="c", subcore_axis_name="s", num_cores=NC, num_subcores=NS
    ),
    compiler_params=pltpu.CompilerParams(needs_layout_passes=False),
    scratch_types=[
        pltpu.VMEM((RPT, K), jnp.int32),
        pltpu.VMEM((RPT, K), jnp.float32),
        pltpu.VMEM((C, IL), jnp.float32),
        pltpu.VMEM((C, IL), jnp.float32),
        pltpu.SemaphoreType.DMA,
        pltpu.SemaphoreType.DMA,
    ],
)


BO = 512


def _mm_body(x_ref, mt_ref, b_ref, o_ref):
    acc = lax.dot_general(
        x_ref[...].astype(jnp.bfloat16), mt_ref[...].astype(jnp.bfloat16),
        (((1,), (1,)), ((), ())),
        preferred_element_type=jnp.float32,
    )
    o_ref[...] = acc + b_ref[...]


def _matmul(x, mt, bias2d):
    return pl.pallas_call(
        _mm_body,
        grid=(OS // BO,),
        in_specs=[
            pl.BlockSpec((B, IL), lambda i: (0, 0)),
            pl.BlockSpec((BO, IL), lambda i: (i, 0)),
            pl.BlockSpec((1, BO), lambda i: (0, i)),
        ],
        out_specs=pl.BlockSpec((B, BO), lambda i: (0, i)),
        out_shape=jax.ShapeDtypeStruct((B, OS), jnp.float32),
    )(x, mt, bias2d)


def kernel(input, weight, bias, indx_seqs):
    mt = _build_mt(weight, indx_seqs)
    return _matmul(input, mt, bias.reshape(1, O))


# speedup vs baseline: 1.2042x; 1.0103x over previous
import functools

import jax
import jax.numpy as jnp
from jax import lax
from jax.experimental import pallas as pl
from jax.experimental.pallas import tpu as pltpu
from jax.experimental.pallas import tpu_sc as plsc

O = 2048
IL = 4096
K = 32
B = 1024

NC = 2
NS = 16
NW = NC * NS
NSTAGE = 1
OS = O // NSTAGE
RPT = OS // NW
C = 4
NCH = RPT // C


def _sc_body(w_hbm, idx_hbm, mt_hbm, idx_v, w_v, buf0, buf1, sem0, sem1):
    wid = lax.axis_index("c") * NS + lax.axis_index("s")
    rbase = wid * RPT
    hi = pltpu.async_copy(idx_hbm.at[pl.ds(rbase, RPT)], idx_v, sem0)
    hw = pltpu.async_copy(w_hbm.at[pl.ds(rbase, RPT)], w_v, sem1)

    z16 = jnp.zeros((16,), jnp.float32)
    bufs = (buf0, buf1)
    sems = (sem0, sem1)

    def _memset(t, carry):
        for j in range(C):
            buf0[j, pl.ds(t * 16, 16)] = z16
            buf1[j, pl.ds(t * 16, 16)] = z16
        return carry
    lax.fori_loop(0, IL // 16, _memset, 0)
    hi.wait()
    hw.wait()

    def _for_chunk(c, fn):
        b = bufs[c % 2]
        for j in range(C):
            r = c * C + j
            jv = jnp.full((16,), j, jnp.int32)
            for h in range(2):
                fn(b, [jv, idx_v[r, pl.ds(h * 16, 16)]], w_v[r, pl.ds(h * 16, 16)])

    handles = [None, None]
    for c in range(NCH):
        s = c % 2
        if handles[s] is not None:
            handles[s].wait()
            _for_chunk(c - 2, lambda b, ix, wv: plsc.store_scatter(b, ix, z16))
        _for_chunk(c, lambda b, ix, wv: plsc.addupdate_scatter(b, ix, wv))
        dst = mt_hbm.at[pl.ds(rbase + c * C, C)]
        handles[s] = pltpu.async_copy(bufs[s], dst, sems[s])
    handles[0].wait()
    handles[1].wait()


_build_mt = pl.kernel(
    _sc_body,
    out_type=jax.ShapeDtypeStruct((OS, IL), jnp.float32),
    mesh=plsc.VectorSubcoreMesh(
        core_axis_name="c", subcore_axis_name="s", num_cores=NC, num_subcores=NS
    ),
    compiler_params=pltpu.CompilerParams(needs_layout_passes=False),
    scratch_types=[
        pltpu.VMEM((RPT, K), jnp.int32),
        pltpu.VMEM((RPT, K), jnp.float32),
        pltpu.VMEM((C, IL), jnp.float32),
        pltpu.VMEM((C, IL), jnp.float32),
        pltpu.SemaphoreType.DMA,
        pltpu.SemaphoreType.DMA,
    ],
)


BO = 256


def _mm_body(x_ref, mt_ref, b_ref, o_ref):
    acc = lax.dot_general(
        x_ref[...].astype(jnp.bfloat16), mt_ref[...].astype(jnp.bfloat16),
        (((1,), (1,)), ((), ())),
        preferred_element_type=jnp.float32,
    )
    o_ref[...] = acc + b_ref[...]


def _matmul(x, mt, bias2d):
    return pl.pallas_call(
        _mm_body,
        grid=(OS // BO,),
        in_specs=[
            pl.BlockSpec((B, IL), lambda i: (0, 0)),
            pl.BlockSpec((BO, IL), lambda i: (i, 0)),
            pl.BlockSpec((1, BO), lambda i: (0, i)),
        ],
        out_specs=pl.BlockSpec((B, BO), lambda i: (0, i)),
        out_shape=jax.ShapeDtypeStruct((B, OS), jnp.float32),
    )(x, mt, bias2d)


def kernel(input, weight, bias, indx_seqs):
    mt = _build_mt(weight, indx_seqs)
    return _matmul(input, mt, bias.reshape(1, O))
